# baseline (device time: 27450 ns/iter reference)
import jax
import jax.numpy as jnp
from jax import lax
from jax.experimental import pallas as pl
from jax.experimental.pallas import tpu as pltpu

N_DEV = 32
LOG2_N = 5


def kernel(x, router_W, route_idx, expert_W):
    del router_W
    n_tok, d_model = x.shape
    n_exp_local, _, d_out = expert_W.shape

    def body(x_ref, idx_ref, ew_ref, out_ref, acc_ref, comm_ref,
             send_sems, recv_sems):
        my = lax.axis_index("i")

        mz = my // 8
        mm = my % 8
        my_y = mm // 2
        mx = (mm % 2) ^ (my_y & 1)

        def ring(x_, y_, z_):
            return z_ * 8 + y_ * 2 + (x_ ^ (y_ & 1))

        partners = [
            ring(mx ^ 1, my_y, mz),
            ring(mx, my_y ^ 1, mz),
            ring(mx, my_y, mz ^ 1),
            ring(mx, my_y ^ 2, mz),
            ring(mx, my_y, mz ^ 2),
        ]

        barrier = pltpu.get_barrier_semaphore()
        for r in range(LOG2_N):
            pl.semaphore_signal(
                barrier, inc=1,
                device_id=(partners[r],),
                device_id_type=pl.DeviceIdType.MESH,
            )
        pl.semaphore_wait(barrier, LOG2_N)

        xb = x_ref[...].astype(jnp.bfloat16)
        idx = idx_ref[...]
        e0 = my * n_exp_local
        acc = jnp.zeros((n_tok, d_out), jnp.float32)
        for e in range(n_exp_local):
            w = ew_ref[e].astype(jnp.bfloat16)
            h = jnp.dot(xb, w, preferred_element_type=jnp.float32)
            acc = acc + jnp.where(idx == e0 + e, h, 0.0)
        acc_ref[0] = acc.astype(jnp.bfloat16)

        rdmas = []
        for r in range(LOG2_N):
            p = r & 1
            partner = partners[r]
            rdma = pltpu.make_async_remote_copy(
                src_ref=acc_ref.at[p],
                dst_ref=comm_ref.at[r],
                send_sem=send_sems.at[r],
                recv_sem=recv_sems.at[r],
                device_id=(partner,),
                device_id_type=pl.DeviceIdType.MESH,
            )
            rdma.start()
            if r > 0:
                rdmas[r - 1].wait_send()
            rdma.wait_recv()
            acc_ref[1 - p] = acc_ref[p] + comm_ref[r]
            rdmas.append(rdma)
        rdmas[-1].wait_send()
        out_ref[...] = acc_ref[LOG2_N & 1].astype(jnp.float32)

    return pl.pallas_call(
        body,
        out_shape=jax.ShapeDtypeStruct((n_tok, d_out), jnp.float32),
        in_specs=[
            pl.BlockSpec(memory_space=pltpu.VMEM),
            pl.BlockSpec(memory_space=pltpu.VMEM),
            pl.BlockSpec(memory_space=pltpu.VMEM),
        ],
        out_specs=pl.BlockSpec(memory_space=pltpu.VMEM),
        scratch_shapes=[
            pltpu.VMEM((2, n_tok, d_out), jnp.bfloat16),
            pltpu.VMEM((LOG2_N, n_tok, d_out), jnp.bfloat16),
            pltpu.SemaphoreType.DMA((LOG2_N,)),
            pltpu.SemaphoreType.DMA((LOG2_N,)),
        ],
        compiler_params=pltpu.CompilerParams(collective_id=0),
    )(x, route_idx, expert_W)


# device time: 20682 ns/iter; 1.3272x vs baseline; 1.3272x over previous
import jax
import jax.numpy as jnp
from jax import lax
from jax.experimental import pallas as pl
from jax.experimental.pallas import tpu as pltpu

N_DEV = 32
LOG2_N = 5

import os as _os
_NROUNDS = int(_os.environ.get("NROUNDS", str(LOG2_N)))
CHUNKS = int(_os.environ.get("CHUNKS", "4"))


def kernel(x, router_W, route_idx, expert_W):
    del router_W
    n_tok, d_model = x.shape
    n_exp_local, _, d_out = expert_W.shape

    def body(x_ref, idx_ref, ew_ref, out_ref, acc_ref, comm_ref,
             send_sems, recv_sems):
        my = lax.axis_index("i")

        mz = my // 8
        mm = my % 8
        my_y = mm // 2
        mx = (mm % 2) ^ (my_y & 1)

        def ring(x_, y_, z_):
            return z_ * 8 + y_ * 2 + (x_ ^ (y_ & 1))

        partners = [
            ring(mx ^ 1, my_y, mz),
            ring(mx, my_y ^ 1, mz),
            ring(mx, my_y, mz ^ 1),
            ring(mx, my_y ^ 2, mz),
            ring(mx, my_y, mz ^ 2),
        ]
        rots = [0, 2, 4, 1, 3]
        orders = [
            [(r + rots[c % 5]) % LOG2_N for r in range(LOG2_N)]
            for c in range(CHUNKS)
        ]

        barrier = pltpu.get_barrier_semaphore()
        for r in range(LOG2_N):
            pl.semaphore_signal(
                barrier, inc=1,
                device_id=(partners[r],),
                device_id_type=pl.DeviceIdType.MESH,
            )

        xb = x_ref[...].astype(jnp.bfloat16)
        idx = idx_ref[...]
        e0 = my * n_exp_local
        xm = jnp.concatenate(
            [jnp.where(idx == e0 + e, xb, 0) for e in range(n_exp_local)],
            axis=1,
        )
        wc = ew_ref[...].astype(jnp.bfloat16).reshape(
            n_exp_local * d_model, d_out
        )
        h = jnp.dot(xm, wc, preferred_element_type=jnp.float32)
        acc_ref[0] = h.astype(jnp.bfloat16)

        pl.semaphore_wait(barrier, LOG2_N)

        half = n_tok // CHUNKS
        rows = [pl.ds(c * half, half) for c in range(CHUNKS)]

        def mk(r, c, start):
            p = r & 1
            rdma = pltpu.make_async_remote_copy(
                src_ref=acc_ref.at[p, rows[c]],
                dst_ref=comm_ref.at[r, c],
                send_sem=send_sems.at[r, c],
                recv_sem=recv_sems.at[r, c],
                device_id=(partners[orders[c][r]],),
                device_id_type=pl.DeviceIdType.MESH,
            )
            if start:
                rdma.start()
            return rdma

        rdmas = {}
        if _NROUNDS:
            for c in range(CHUNKS):
                rdmas[(0, c)] = mk(0, c, True)
        last = _NROUNDS - 1
        for r in range(_NROUNDS):
            p = r & 1
            for c in range(CHUNKS):
                rdmas[(r, c)].wait_recv()
                if r > 0:
                    rdmas[(r - 1, c)].wait_send()
                s = acc_ref[p, rows[c]] + comm_ref[r, c]
                if r == last:
                    out_ref[rows[c]] = s
                else:
                    acc_ref[1 - p, rows[c]] = s
                    rdmas[(r + 1, c)] = mk(r + 1, c, True)
        if _NROUNDS:
            for c in range(CHUNKS):
                rdmas[(last, c)].wait_send()
        else:
            out_ref[...] = acc_ref[0]

    return pl.pallas_call(
        body,
        out_shape=jax.ShapeDtypeStruct((n_tok, d_out), jnp.bfloat16),
        in_specs=[
            pl.BlockSpec(memory_space=pltpu.VMEM),
            pl.BlockSpec(memory_space=pltpu.VMEM),
            pl.BlockSpec(memory_space=pltpu.VMEM),
        ],
        out_specs=pl.BlockSpec(memory_space=pltpu.VMEM),
        scratch_shapes=[
            pltpu.VMEM((2, n_tok, d_out), jnp.bfloat16),
            pltpu.VMEM(
                (LOG2_N, CHUNKS, n_tok // CHUNKS, d_out), jnp.bfloat16
            ),
            pltpu.SemaphoreType.DMA((LOG2_N, CHUNKS)),
            pltpu.SemaphoreType.DMA((LOG2_N, CHUNKS)),
        ],
        compiler_params=pltpu.CompilerParams(collective_id=0),
    )(x, route_idx, expert_W)
